# baseline (speedup 1.0000x reference)
import functools

import jax
import jax.numpy as jnp
from jax import lax
from jax.experimental import pallas as pl
from jax.experimental.pallas import tpu as pltpu
from jax.experimental.pallas import tpu_sc as plsc

_SPECS = (
    [(1000, i, 1) for i in range(18)]
    + [(100000, i, 1) for i in range(18, 24)]
    + [(100000, 24, 20), (100000, 44, 20)]
)
_B = 4096
_D = 64
_NW = 32
_CHUNK = _B // _NW
_L = 16
_NG = _CHUNK // _L


def _emb_body(ints_hbm, *rest):
    tables = rest[:26]
    out_hbm = rest[26]
    ints_v, idx_v, rows_v, bag_v, acc_v, sem = rest[27:]

    cid = lax.axis_index("c")
    sid = lax.axis_index("s")
    wid = sid * 2 + cid
    base = wid * _CHUNK

    pltpu.sync_copy(ints_hbm.at[:, pl.ds(base, _CHUNK)], ints_v)

    def _build_idx(col, vocab):
        for g in range(_NG):
            sl = pl.ds(g * _L, _L)
            v = ints_v[col, sl]
            v = jnp.minimum(jnp.maximum(v, 0), vocab)
            idx_v[sl] = v

    for s_idx in range(24):
        vocab, off, _len = _SPECS[s_idx]
        _build_idx(off, vocab)
        pltpu.async_copy(tables[s_idx].at[idx_v], rows_v, sem).wait()
        pltpu.sync_copy(rows_v, out_hbm.at[pl.ds(base, _CHUNK), s_idx])

    for s_idx in (24, 25):
        vocab, off, blen = _SPECS[s_idx]
        table = tables[s_idx]

        _build_idx(off, vocab)
        pltpu.async_copy(table.at[idx_v], acc_v, sem).wait()

        def _jbody(j, _, off=off, vocab=vocab, table=table):
            for g in range(_NG):
                sl = pl.ds(g * _L, _L)
                v = ints_v[off + j, sl]
                v = jnp.minimum(jnp.maximum(v, 0), vocab)
                idx_v[sl] = v
            pltpu.async_copy(table.at[idx_v], bag_v, sem).wait()

            def _rbody(r, _):
                for dd in range(_D // _L):
                    x = bag_v[r, pl.ds(dd * _L, _L)]
                    plsc.addupdate(acc_v.at[r, pl.ds(dd * _L, _L)], x)
                return 0

            lax.fori_loop(0, _CHUNK, _rbody, 0)
            return 0

        lax.fori_loop(1, blen, _jbody, 0)

        inv = jnp.float32(1.0 / blen)

        def _sbody(r, _):
            for dd in range(_D // _L):
                sl = pl.ds(dd * _L, _L)
                acc_v[r, sl] = acc_v[r, sl] * inv
            return 0

        lax.fori_loop(0, _CHUNK, _sbody, 0)
        pltpu.sync_copy(acc_v, out_hbm.at[pl.ds(base, _CHUNK), s_idx])


@functools.partial(jax.jit, static_argnames=())
def kernel(int_feats, tables):
    ints_t = jnp.transpose(int_feats)
    call = pl.kernel(
        _emb_body,
        out_type=jax.ShapeDtypeStruct((_B, 26, _D), jnp.float32),
        mesh=plsc.VectorSubcoreMesh(core_axis_name="c", subcore_axis_name="s"),
        compiler_params=pltpu.CompilerParams(use_tc_tiling_on_sc=False),
        scratch_types=[
            pltpu.VMEM((64, _CHUNK), jnp.int32),
            pltpu.VMEM((_CHUNK,), jnp.int32),
            pltpu.VMEM((_CHUNK, _D), jnp.float32),
            pltpu.VMEM((_CHUNK, _D), jnp.float32),
            pltpu.VMEM((_CHUNK, _D), jnp.float32),
            pltpu.SemaphoreType.DMA,
        ],
    )
    return call(ints_t, *tables)
